# baseline (speedup 1.0000x reference)
import functools

import jax
import jax.numpy as jnp
from jax import lax
from jax.experimental import pallas as pl
from jax.experimental.pallas import tpu as pltpu
from jax.experimental.pallas import tpu_sc as plsc

_IB = 1408


def _ffn_kernel(x_ref, gate_ref, down_ref, up_ref, y_ref):
    j = pl.program_id(1)
    x = x_ref[...]
    g = lax.dot_general(x, gate_ref[0], (((1,), (1,)), ((), ())),
                        preferred_element_type=jnp.float32)
    d = lax.dot_general(x, down_ref[0], (((1,), (1,)), ((), ())),
                        preferred_element_type=jnp.float32)
    h = (g * jax.nn.sigmoid(g)) * d
    p = lax.dot_general(h, up_ref[0], (((1,), (1,)), ((), ())),
                        preferred_element_type=jnp.float32)

    @pl.when(j == 0)
    def _():
        y_ref[...] = jnp.zeros_like(y_ref)

    y_ref[...] += p[None]


def _dense_expert_outputs(x, gate_proj, up_proj, down_proj):
    T, H = x.shape
    E, I, _ = gate_proj.shape
    return pl.pallas_call(
        _ffn_kernel,
        grid=(E, I // _IB),
        in_specs=[
            pl.BlockSpec((T, H), lambda e, j: (0, 0)),
            pl.BlockSpec((1, _IB, H), lambda e, j: (e, j, 0)),
            pl.BlockSpec((1, _IB, H), lambda e, j: (e, j, 0)),
            pl.BlockSpec((1, H, _IB), lambda e, j: (e, 0, j)),
        ],
        out_specs=pl.BlockSpec((1, T, H), lambda e, j: (e, 0, 0)),
        out_shape=jax.ShapeDtypeStruct((E, T, H), jnp.float32),
    )(x, gate_proj, down_proj, up_proj)


_N_WORKERS = 4
_ROWS_PER_WORKER = 8


def _make_sc_gather(P, H):
    mesh = plsc.VectorSubcoreMesh(core_axis_name="c", subcore_axis_name="s")

    @functools.partial(
        pl.kernel, mesh=mesh,
        out_type=jax.ShapeDtypeStruct((P, H), jnp.float32),
        scratch_types=[
            pltpu.VMEM((_ROWS_PER_WORKER,), jnp.int32),
            pltpu.VMEM((_ROWS_PER_WORKER, H), jnp.float32),
            pltpu.SemaphoreType.DMA,
        ],
    )
    def sc_gather(y_hbm, ridx_hbm, out_hbm, idx_v, rows_v, sem):
        wid = lax.axis_index("s") * 2 + lax.axis_index("c")

        @pl.when(wid < _N_WORKERS)
        def _():
            base = wid * _ROWS_PER_WORKER
            pltpu.sync_copy(ridx_hbm.at[pl.ds(base, _ROWS_PER_WORKER)], idx_v)
            pltpu.async_copy(y_hbm.at[idx_v], rows_v, sem).wait()
            pltpu.sync_copy(rows_v, out_hbm.at[pl.ds(base, _ROWS_PER_WORKER)])

    return sc_gather


def kernel(x, expert_indices, gate_proj, up_proj, down_proj):
    T, H = x.shape
    A = expert_indices.shape[1]
    E, I, _ = gate_proj.shape
    P = T * A

    y = _dense_expert_outputs(x, gate_proj, up_proj, down_proj)

    idx_flat = expert_indices.reshape(-1).astype(jnp.int32)
    ridx = idx_flat * T + (jnp.arange(P, dtype=jnp.int32) // A)

    out = _make_sc_gather(P, H)(y.reshape(E * T, H), ridx)
    return out.reshape(T, A, H)
